# baseline (speedup 1.0000x reference)
import jax
import jax.numpy as jnp
import numpy as np
from jax.experimental import pallas as pl
from jax.experimental.pallas import tpu as pltpu

_ALPHA = 300.0
_N_ATTR = 16
_P = 4096


def _interp_matrix_t(old: int, new: int) -> np.ndarray:
    pos = np.arange(new, dtype=np.float32) * np.float32((old - 1) / (new - 1))
    lo = np.floor(pos).astype(np.int32)
    hi = np.minimum(lo + 1, old - 1)
    w = pos - lo.astype(np.float32)
    m = np.zeros((new, old), dtype=np.float32)
    m[np.arange(new), lo] += (np.float32(1.0) - w)
    m[np.arange(new), hi] += w
    return np.ascontiguousarray(m.T)


def _proj_kernel(w1_ref, e_ref, v_ref):
    v_ref[0] = jnp.dot(w1_ref[...], e_ref[0],
                       preferred_element_type=jnp.float32)


def _resize_chain(v, lht, lwt, ch):
    vt = jnp.swapaxes(v, 1, 2)
    eh = jnp.dot(vt.reshape(ch * 64, 64), lht,
                 preferred_element_type=jnp.float32)
    ehw = jnp.swapaxes(eh.reshape(ch, 64, 128), 1, 2)
    return jnp.dot(ehw.reshape(ch * 128, 64), lwt,
                   preferred_element_type=jnp.float32)


def _resize_kernel(v_ref, bpv_ref, lht_ref, lwt_ref, u1_ref, bc_ref):
    lht = lht_ref[...]
    lwt = lwt_ref[...]
    u1_ref[0] = _resize_chain(v_ref[0], lht, lwt, 128)
    bc_ref[0] = _resize_chain(bpv_ref[0], lht, lwt, 64)


def _main_kernel(x_ref, u1_ref, bc_ref, w1_ref, b1_ref, w2_ref, b2_ref,
                 o_ref):
    h1 = jnp.dot(w1_ref[...], x_ref[0],
                 preferred_element_type=jnp.float32)
    hidd = jnp.maximum(h1 + u1_ref[0] + b1_ref[...], 0.0)
    a1 = jnp.dot(w2_ref[...], hidd,
                 preferred_element_type=jnp.float32)
    z = a1 + b2_ref[...]
    attr = jnp.maximum(z, 0.0) + jnp.log1p(jnp.exp(-jnp.abs(z)))

    bc = bc_ref[0]
    s = jnp.float32(np.sqrt(_ALPHA))
    attrs = attr * s
    bcs = bc * s
    acc = jnp.zeros_like(bc)
    for a in range(_N_ATTR):
        dxp = attrs[a:a + 1] - bcs
        acc = acc + dxp / (1.0 + dxp * dxp)
    o_ref[0] = bc + acc * jnp.float32(1.0 / np.sqrt(_ALPHA))


@jax.jit
def kernel(x, b_prev, prev_b_embedding, w1, b1, w2, b2):
    n, c, h, w = x.shape
    nb = b_prev.shape[1]
    md = w1.shape[0]
    na = w2.shape[0]
    hw = h * w
    grid_t = hw // _P

    lht = jnp.asarray(_interp_matrix_t(64, h))
    lwt = jnp.asarray(_interp_matrix_t(64, w))

    emb2 = prev_b_embedding.reshape(n, c, 64 * 64)
    v = pl.pallas_call(
        _proj_kernel,
        grid=(n,),
        in_specs=[
            pl.BlockSpec((md, c), lambda i: (0, 0)),
            pl.BlockSpec((1, c, 64 * 64), lambda i: (i, 0, 0)),
        ],
        out_specs=pl.BlockSpec((1, md, 64 * 64), lambda i: (i, 0, 0)),
        out_shape=jax.ShapeDtypeStruct((n, md, 64 * 64), jnp.float32),
        compiler_params=pltpu.CompilerParams(
            dimension_semantics=("arbitrary",)),
    )(w1, emb2)

    v4 = v.reshape(n, md, 64, 64)
    u1f, bcf = pl.pallas_call(
        _resize_kernel,
        grid=(n,),
        in_specs=[
            pl.BlockSpec((1, md, 64, 64), lambda i: (i, 0, 0, 0)),
            pl.BlockSpec((1, nb, 64, 64), lambda i: (i, 0, 0, 0)),
            pl.BlockSpec((64, 128), lambda i: (0, 0)),
            pl.BlockSpec((64, 128), lambda i: (0, 0)),
        ],
        out_specs=[
            pl.BlockSpec((1, md * h, w), lambda i: (i, 0, 0)),
            pl.BlockSpec((1, nb * h, w), lambda i: (i, 0, 0)),
        ],
        out_shape=[
            jax.ShapeDtypeStruct((n, md * h, w), jnp.float32),
            jax.ShapeDtypeStruct((n, nb * h, w), jnp.float32),
        ],
        compiler_params=pltpu.CompilerParams(
            dimension_semantics=("arbitrary",),
            vmem_limit_bytes=64 * 1024 * 1024,
        ),
    )(v4, b_prev, lht, lwt)

    x2 = x.reshape(n, c, hw)
    u12 = u1f.reshape(n, md, hw)
    bc2 = bcf.reshape(n, nb, hw)
    out = pl.pallas_call(
        _main_kernel,
        grid=(n, grid_t),
        in_specs=[
            pl.BlockSpec((1, c, _P), lambda i, t: (i, 0, t)),
            pl.BlockSpec((1, md, _P), lambda i, t: (i, 0, t)),
            pl.BlockSpec((1, nb, _P), lambda i, t: (i, 0, t)),
            pl.BlockSpec((md, c), lambda i, t: (0, 0)),
            pl.BlockSpec((md, 1), lambda i, t: (0, 0)),
            pl.BlockSpec((na, md), lambda i, t: (0, 0)),
            pl.BlockSpec((na, 1), lambda i, t: (0, 0)),
        ],
        out_specs=pl.BlockSpec((1, nb, _P), lambda i, t: (i, 0, t)),
        out_shape=jax.ShapeDtypeStruct((n, nb, hw), jnp.float32),
        compiler_params=pltpu.CompilerParams(
            dimension_semantics=("parallel", "arbitrary"),
            vmem_limit_bytes=64 * 1024 * 1024,
        ),
    )(x2, u12, bc2, w1, b1[:, None], w2, b2[:, None])

    out = out.reshape(n, nb, h, w)
    return (out, out)


# speedup vs baseline: 1.5972x; 1.5972x over previous
import jax
import jax.numpy as jnp
import numpy as np
from jax.experimental import pallas as pl
from jax.experimental.pallas import tpu as pltpu

_ALPHA = 300.0
_N_ATTR = 16
_R = 32


def _interp_matrix_t(old: int, new: int) -> np.ndarray:
    pos = np.arange(new, dtype=np.float32) * np.float32((old - 1) / (new - 1))
    lo = np.floor(pos).astype(np.int32)
    hi = np.minimum(lo + 1, old - 1)
    w = pos - lo.astype(np.float32)
    m = np.zeros((new, old), dtype=np.float32)
    m[np.arange(new), lo] += (np.float32(1.0) - w)
    m[np.arange(new), hi] += w
    return np.ascontiguousarray(m.T)


def _resize3(v, lht, lwt, ch):
    vt = jnp.swapaxes(v, 1, 2)
    eh = jnp.dot(vt.reshape(ch * 64, 64), lht,
                 preferred_element_type=jnp.float32)
    ehw = jnp.swapaxes(eh.reshape(ch, 64, 128), 1, 2)
    ew = jnp.dot(ehw.reshape(ch * 128, 64), lwt,
                 preferred_element_type=jnp.float32)
    return ew.reshape(ch, 128, 128)


def _fused_kernel(x_ref, emb_ref, bpv_ref, lht_ref, lwt_ref, w1_ref, b1_ref,
                  w2_ref, b2_ref, out_ref, er_ref, bc_ref):
    t = pl.program_id(1)

    @pl.when(t == 0)
    def _prep():
        lht = lht_ref[...]
        lwt = lwt_ref[...]
        for cc in range(4):
            v = emb_ref[0, cc * 64:(cc + 1) * 64]
            er_ref[cc * 64:(cc + 1) * 64] = _resize3(v, lht, lwt, 64)
        bc_ref[...] = _resize3(bpv_ref[0], lht, lwt, 64)

    xe = x_ref[0] + er_ref[:, pl.ds(t * _R, _R), :]
    h1 = jax.lax.dot_general(w1_ref[...], xe, (((1,), (0,)), ((), ())),
                             preferred_element_type=jnp.float32)
    hidd = jnp.maximum(h1 + b1_ref[...], 0.0)
    a1 = jax.lax.dot_general(w2_ref[...], hidd, (((1,), (0,)), ((), ())),
                             preferred_element_type=jnp.float32)
    z = a1 + b2_ref[...]
    attr = jnp.maximum(z, 0.0) + jnp.log1p(jnp.exp(-jnp.abs(z)))

    bc = bc_ref[:, pl.ds(t * _R, _R), :]
    s = jnp.float32(np.sqrt(_ALPHA))
    attrs = attr * s
    bcs = bc * s
    acc = jnp.zeros_like(bc)
    for a in range(_N_ATTR):
        dxp = attrs[a:a + 1] - bcs
        acc = acc + dxp / (1.0 + dxp * dxp)
    out_ref[0] = bc + acc * jnp.float32(1.0 / np.sqrt(_ALPHA))


@jax.jit
def kernel(x, b_prev, prev_b_embedding, w1, b1, w2, b2):
    n, c, h, w = x.shape
    nb = b_prev.shape[1]
    md = w1.shape[0]
    na = w2.shape[0]
    grid_t = h // _R

    lht = jnp.asarray(_interp_matrix_t(64, h))
    lwt = jnp.asarray(_interp_matrix_t(64, w))
    b1b = jnp.broadcast_to(b1[:, None, None], (md, 1, w))
    b2b = jnp.broadcast_to(b2[:, None, None], (na, 1, w))

    out = pl.pallas_call(
        _fused_kernel,
        grid=(n, grid_t),
        in_specs=[
            pl.BlockSpec((1, c, _R, w), lambda i, t: (i, 0, t, 0)),
            pl.BlockSpec((1, c, 64, 64), lambda i, t: (i, 0, 0, 0)),
            pl.BlockSpec((1, nb, 64, 64), lambda i, t: (i, 0, 0, 0)),
            pl.BlockSpec((64, 128), lambda i, t: (0, 0)),
            pl.BlockSpec((64, 128), lambda i, t: (0, 0)),
            pl.BlockSpec((md, c), lambda i, t: (0, 0)),
            pl.BlockSpec((md, 1, w), lambda i, t: (0, 0, 0)),
            pl.BlockSpec((na, md), lambda i, t: (0, 0)),
            pl.BlockSpec((na, 1, w), lambda i, t: (0, 0, 0)),
        ],
        out_specs=pl.BlockSpec((1, nb, _R, w), lambda i, t: (i, 0, t, 0)),
        out_shape=jax.ShapeDtypeStruct((n, nb, h, w), jnp.float32),
        scratch_shapes=[
            pltpu.VMEM((c, h, w), jnp.float32),
            pltpu.VMEM((nb, h, w), jnp.float32),
        ],
        compiler_params=pltpu.CompilerParams(
            dimension_semantics=("parallel", "arbitrary"),
            vmem_limit_bytes=64 * 1024 * 1024,
        ),
    )(x, prev_b_embedding, b_prev, lht, lwt, w1, b1b, w2, b2b)
    return (out, out)


# speedup vs baseline: 1.6414x; 1.0277x over previous
import jax
import jax.numpy as jnp
import numpy as np
from jax.experimental import pallas as pl
from jax.experimental.pallas import tpu as pltpu

_ALPHA = 300.0
_N_ATTR = 16
_R = 64


def _interp_matrix_t(old: int, new: int) -> np.ndarray:
    pos = np.arange(new, dtype=np.float32) * np.float32((old - 1) / (new - 1))
    lo = np.floor(pos).astype(np.int32)
    hi = np.minimum(lo + 1, old - 1)
    w = pos - lo.astype(np.float32)
    m = np.zeros((new, old), dtype=np.float32)
    m[np.arange(new), lo] += (np.float32(1.0) - w)
    m[np.arange(new), hi] += w
    return np.ascontiguousarray(m.T)


def _resize3(v, lht, lwt, ch):
    vt = jnp.swapaxes(v, 1, 2)
    eh = jnp.dot(vt.reshape(ch * 64, 64), lht,
                 preferred_element_type=jnp.float32)
    ehw = jnp.swapaxes(eh.reshape(ch, 64, 128), 1, 2)
    ew = jnp.dot(ehw.reshape(ch * 128, 64), lwt,
                 preferred_element_type=jnp.float32)
    return ew.reshape(ch, 128, 128)


def _fused_kernel(x_ref, emb_ref, bpv_ref, lht_ref, lwt_ref, w1_ref, b1_ref,
                  w2_ref, b2_ref, out_ref, er_ref, bc_ref):
    t = pl.program_id(1)

    @pl.when(t == 0)
    def _prep():
        lht = lht_ref[...]
        lwt = lwt_ref[...]
        for cc in range(4):
            v = emb_ref[0, cc * 64:(cc + 1) * 64]
            er_ref[cc * 64:(cc + 1) * 64] = _resize3(
                v, lht, lwt, 64).astype(jnp.bfloat16)
        bc_ref[...] = _resize3(bpv_ref[0], lht, lwt, 64)

    er = er_ref[:, pl.ds(t * _R, _R), :].astype(jnp.float32)
    xe = x_ref[0] + er
    h1 = jax.lax.dot_general(w1_ref[...], xe, (((1,), (0,)), ((), ())),
                             preferred_element_type=jnp.float32)
    hidd = jnp.maximum(h1 + b1_ref[...], 0.0)
    a1 = jax.lax.dot_general(w2_ref[...], hidd, (((1,), (0,)), ((), ())),
                             preferred_element_type=jnp.float32)
    z = a1 + b2_ref[...]
    attr = jnp.maximum(z, 0.0) + jnp.log1p(jnp.exp(-jnp.abs(z)))

    bc = bc_ref[:, pl.ds(t * _R, _R), :]
    s = jnp.float32(np.sqrt(_ALPHA))
    attrs = attr * s
    bcs = bc * s
    acc = jnp.zeros_like(bc)
    for a in range(_N_ATTR):
        dxp = attrs[a:a + 1] - bcs
        acc = acc + dxp / (1.0 + dxp * dxp)
    out_ref[0] = bc + acc * jnp.float32(1.0 / np.sqrt(_ALPHA))


@jax.jit
def kernel(x, b_prev, prev_b_embedding, w1, b1, w2, b2):
    n, c, h, w = x.shape
    nb = b_prev.shape[1]
    md = w1.shape[0]
    na = w2.shape[0]
    grid_t = h // _R

    lht = jnp.asarray(_interp_matrix_t(64, h))
    lwt = jnp.asarray(_interp_matrix_t(64, w))
    b1b = jnp.broadcast_to(b1[:, None, None], (md, 1, w))
    b2b = jnp.broadcast_to(b2[:, None, None], (na, 1, w))

    out = pl.pallas_call(
        _fused_kernel,
        grid=(n, grid_t),
        in_specs=[
            pl.BlockSpec((1, c, _R, w), lambda i, t: (i, 0, t, 0)),
            pl.BlockSpec((1, c, 64, 64), lambda i, t: (i, 0, 0, 0)),
            pl.BlockSpec((1, nb, 64, 64), lambda i, t: (i, 0, 0, 0)),
            pl.BlockSpec((64, 128), lambda i, t: (0, 0)),
            pl.BlockSpec((64, 128), lambda i, t: (0, 0)),
            pl.BlockSpec((md, c), lambda i, t: (0, 0)),
            pl.BlockSpec((md, 1, w), lambda i, t: (0, 0, 0)),
            pl.BlockSpec((na, md), lambda i, t: (0, 0)),
            pl.BlockSpec((na, 1, w), lambda i, t: (0, 0, 0)),
        ],
        out_specs=pl.BlockSpec((1, nb, _R, w), lambda i, t: (i, 0, t, 0)),
        out_shape=jax.ShapeDtypeStruct((n, nb, h, w), jnp.float32),
        scratch_shapes=[
            pltpu.VMEM((c, h, w), jnp.bfloat16),
            pltpu.VMEM((nb, h, w), jnp.float32),
        ],
        compiler_params=pltpu.CompilerParams(
            dimension_semantics=("parallel", "arbitrary"),
            vmem_limit_bytes=64 * 1024 * 1024,
        ),
    )(x, prev_b_embedding, b_prev, lht, lwt, w1, b1b, w2, b2b)
    return (out, out)


# speedup vs baseline: 1.7779x; 1.0832x over previous
import jax
import jax.numpy as jnp
import numpy as np
from jax.experimental import pallas as pl
from jax.experimental.pallas import tpu as pltpu

_ALPHA = 300.0
_N_ATTR = 16
_R = 64


def _interp_matrix_t(old: int, new: int) -> np.ndarray:
    pos = np.arange(new, dtype=np.float32) * np.float32((old - 1) / (new - 1))
    lo = np.floor(pos).astype(np.int32)
    hi = np.minimum(lo + 1, old - 1)
    w = pos - lo.astype(np.float32)
    m = np.zeros((new, old), dtype=np.float32)
    m[np.arange(new), lo] += (np.float32(1.0) - w)
    m[np.arange(new), hi] += w
    return np.ascontiguousarray(m.T)


def _resize3(v, lht, lwt, ch):
    vt = jnp.swapaxes(v, 1, 2)
    eh = jnp.dot(vt.reshape(ch * 64, 64), lht,
                 preferred_element_type=jnp.float32)
    ehw = jnp.swapaxes(eh.reshape(ch, 64, 128), 1, 2)
    ew = jnp.dot(ehw.reshape(ch * 128, 64), lwt,
                 preferred_element_type=jnp.float32)
    return ew.reshape(ch, 128, 128)


def _fused_kernel(x_ref, emb_ref, bpv_ref, lht_ref, lwt_ref, w1_ref, b1_ref,
                  w2_ref, b2_ref, out_ref, er_ref, bc_ref):
    t = pl.program_id(1)

    @pl.when(t == 0)
    def _prep():
        lht = lht_ref[...]
        lwt = lwt_ref[...]
        for cc in range(4):
            v = emb_ref[0, cc * 64:(cc + 1) * 64]
            er_ref[cc * 64:(cc + 1) * 64] = _resize3(
                v, lht, lwt, 64).astype(jnp.bfloat16)
        bc_ref[...] = _resize3(bpv_ref[0], lht, lwt, 64)

    er = er_ref[:, pl.ds(t * _R, _R), :].astype(jnp.float32)
    xe = x_ref[0] + er
    h1 = jax.lax.dot_general(w1_ref[...], xe, (((1,), (0,)), ((), ())),
                             preferred_element_type=jnp.float32)
    hidd = jnp.maximum(h1 + b1_ref[...], 0.0)
    a1 = jax.lax.dot_general(w2_ref[...], hidd, (((1,), (0,)), ((), ())),
                             preferred_element_type=jnp.float32)
    z = a1 + b2_ref[...]
    attr = jnp.maximum(z, 0.0) + jnp.log1p(jnp.exp(-jnp.abs(z)))

    s = jnp.float32(np.sqrt(_ALPHA))
    inv_s = jnp.float32(1.0 / np.sqrt(_ALPHA))
    attrs = attr * s
    for r0 in range(0, _R, 8):
        attrs_r = attrs[:, r0:r0 + 8, :]
        for j0 in range(0, 64, 8):
            bcs = bc_ref[j0:j0 + 8, pl.ds(t * _R + r0, 8), :] * s
            acc = None
            for a in range(_N_ATTR):
                dxp = attrs_r[a:a + 1] - bcs
                term = dxp / (1.0 + dxp * dxp)
                acc = term if acc is None else acc + term
            out_ref[0, j0:j0 + 8, r0:r0 + 8, :] = (bcs + acc) * inv_s


@jax.jit
def kernel(x, b_prev, prev_b_embedding, w1, b1, w2, b2):
    n, c, h, w = x.shape
    nb = b_prev.shape[1]
    md = w1.shape[0]
    na = w2.shape[0]
    grid_t = h // _R

    lht = jnp.asarray(_interp_matrix_t(64, h))
    lwt = jnp.asarray(_interp_matrix_t(64, w))
    b1b = jnp.broadcast_to(b1[:, None, None], (md, 1, w))
    b2b = jnp.broadcast_to(b2[:, None, None], (na, 1, w))

    out = pl.pallas_call(
        _fused_kernel,
        grid=(n, grid_t),
        in_specs=[
            pl.BlockSpec((1, c, _R, w), lambda i, t: (i, 0, t, 0)),
            pl.BlockSpec((1, c, 64, 64), lambda i, t: (i, 0, 0, 0)),
            pl.BlockSpec((1, nb, 64, 64), lambda i, t: (i, 0, 0, 0)),
            pl.BlockSpec((64, 128), lambda i, t: (0, 0)),
            pl.BlockSpec((64, 128), lambda i, t: (0, 0)),
            pl.BlockSpec((md, c), lambda i, t: (0, 0)),
            pl.BlockSpec((md, 1, w), lambda i, t: (0, 0, 0)),
            pl.BlockSpec((na, md), lambda i, t: (0, 0)),
            pl.BlockSpec((na, 1, w), lambda i, t: (0, 0, 0)),
        ],
        out_specs=pl.BlockSpec((1, nb, _R, w), lambda i, t: (i, 0, t, 0)),
        out_shape=jax.ShapeDtypeStruct((n, nb, h, w), jnp.float32),
        scratch_shapes=[
            pltpu.VMEM((c, h, w), jnp.bfloat16),
            pltpu.VMEM((nb, h, w), jnp.float32),
        ],
        compiler_params=pltpu.CompilerParams(
            dimension_semantics=("parallel", "arbitrary"),
            vmem_limit_bytes=64 * 1024 * 1024,
        ),
    )(x, prev_b_embedding, b_prev, lht, lwt, w1, b1b, w2, b2b)
    return (out, out)
